# baseline (speedup 1.0000x reference)
import jax
import jax.numpy as jnp
from jax import lax
from jax.experimental import pallas as pl
from jax.experimental.pallas import tpu as pltpu
from jax.experimental.pallas import tpu_sc as plsc

B = 160000
D = 256
NUM_RELS = 16
LANES = 16
NW = 32
PER_W = B // NW
RCHUNK = 48
GROUPS = RCHUNK // LANES
NCHUNK = 4992 // RCHUNK
REM = PER_W - NCHUNK * RCHUNK


def _row_score(u_v, v_v, w_v, e_v, row, ub):
    e = e_v[pl.ds(row, LANES)][0]
    wb = e * D
    acc = (u_v[pl.ds(ub, LANES)]
           * w_v[pl.ds(wb, LANES)]
           * v_v[pl.ds(ub, LANES)])
    for k in range(1, D // LANES):
        acc = acc + (u_v[pl.ds(ub + k * LANES, LANES)]
                     * w_v[pl.ds(wb + k * LANES, LANES)]
                     * v_v[pl.ds(ub + k * LANES, LANES)])
    lanes = lax.iota(jnp.int32, LANES)
    for step in (8, 4, 2, 1):
        acc = acc + acc.at[lanes ^ step].get(mode="promise_in_bounds")
    return acc


def _group(u_v, v_v, w_v, e_v, o_v, erow0, brow0, orow0):
    lanes = lax.iota(jnp.int32, LANES)

    def row_body(jj, res):
        s = _row_score(u_v, v_v, w_v, e_v, erow0 + jj, (brow0 + jj) * D)
        return jnp.where(lanes == jj, s, res)

    res = lax.fori_loop(0, LANES, row_body,
                        jnp.zeros((LANES,), jnp.float32), unroll=2)
    o_v[pl.ds(orow0, LANES)] = res


def _body(u_hbm, v_hbm, e_hbm, w_hbm, out_hbm,
          w_v, e_v, o_v, u_v, v_v, sem_u, sem_v):
    wid = lax.axis_index("s") * 2 + lax.axis_index("c")
    base = wid * PER_W

    pltpu.sync_copy(w_hbm, w_v)
    pltpu.sync_copy(e_hbm.at[pl.ds(base, PER_W)], e_v.at[pl.ds(0, PER_W)])
    e_v[pl.ds(PER_W, LANES)] = jnp.zeros((LANES,), jnp.int32)

    def chunk_body(c, _):
        row0 = base + c * RCHUNK
        cu = pltpu.async_copy(u_hbm.at[pl.ds(row0 * D, RCHUNK * D)], u_v, sem_u)
        cv = pltpu.async_copy(v_hbm.at[pl.ds(row0 * D, RCHUNK * D)], v_v, sem_v)
        cu.wait()
        cv.wait()
        for g in range(GROUPS):
            _group(u_v, v_v, w_v, e_v,
                   o_v,
                   c * RCHUNK + g * LANES,
                   g * LANES,
                   c * RCHUNK + g * LANES)
        return ()

    lax.fori_loop(0, NCHUNK, chunk_body, ())

    tail = NCHUNK * RCHUNK
    pltpu.sync_copy(u_hbm.at[pl.ds((base + tail) * D, REM * D)],
                    u_v.at[pl.ds(0, REM * D)])
    pltpu.sync_copy(v_hbm.at[pl.ds((base + tail) * D, REM * D)],
                    v_v.at[pl.ds(0, REM * D)])
    _group(u_v, v_v, w_v, e_v, o_v, tail, 0, tail)

    pltpu.sync_copy(o_v.at[pl.ds(0, PER_W)], out_hbm.at[pl.ds(base, PER_W)])


@jax.jit
def _distmult_sc(u_flat, v_flat, etype_i32, w_flat):
    mesh = plsc.VectorSubcoreMesh(core_axis_name="c", subcore_axis_name="s")
    return pl.kernel(
        _body,
        out_type=jax.ShapeDtypeStruct((B,), jnp.float32),
        mesh=mesh,
        scratch_types=[
            pltpu.VMEM((NUM_RELS * D,), jnp.float32),
            pltpu.VMEM((PER_W + 2 * LANES,), jnp.int32),
            pltpu.VMEM((PER_W + LANES,), jnp.float32),
            pltpu.VMEM((RCHUNK * D,), jnp.float32),
            pltpu.VMEM((RCHUNK * D,), jnp.float32),
            pltpu.SemaphoreType.DMA,
            pltpu.SemaphoreType.DMA,
        ],
    )(u_flat, v_flat, etype_i32, w_flat)


def kernel(u, v, etype_ids, w_relation):
    return _distmult_sc(
        u.reshape(-1),
        v.reshape(-1),
        etype_ids.astype(jnp.int32),
        w_relation.reshape(-1),
    )


# speedup vs baseline: 1.3319x; 1.3319x over previous
import jax
import jax.numpy as jnp
from jax import lax
from jax.experimental import pallas as pl
from jax.experimental.pallas import tpu as pltpu
from jax.experimental.pallas import tpu_sc as plsc

B = 160000
D = 256
NUM_RELS = 16
LANES = 16
NW = 32
PER_W = B // NW
RCHUNK = 48
GROUPS = RCHUNK // LANES
NCHUNK = 4992 // RCHUNK
REM = PER_W - NCHUNK * RCHUNK


def _row_score(u_v, v_v, w_v, e_v, row, ub):
    e = e_v[pl.ds(row, LANES)][0]
    wb = e * D
    acc = (u_v[pl.ds(ub, LANES)]
           * w_v[pl.ds(wb, LANES)]
           * v_v[pl.ds(ub, LANES)])
    for k in range(1, D // LANES):
        acc = acc + (u_v[pl.ds(ub + k * LANES, LANES)]
                     * w_v[pl.ds(wb + k * LANES, LANES)]
                     * v_v[pl.ds(ub + k * LANES, LANES)])
    lanes = lax.iota(jnp.int32, LANES)
    for step in (8, 4, 2, 1):
        acc = acc + acc.at[lanes ^ step].get(mode="promise_in_bounds")
    return acc


def _group(u_v, v_v, w_v, e_v, o_v, erow0, brow0, orow0):
    lanes = lax.iota(jnp.int32, LANES)

    def row_body(jj, res):
        s = _row_score(u_v, v_v, w_v, e_v, erow0 + jj, (brow0 + jj) * D)
        return jnp.where(lanes == jj, s, res)

    res = lax.fori_loop(0, LANES, row_body,
                        jnp.zeros((LANES,), jnp.float32), unroll=2)
    o_v[pl.ds(orow0, LANES)] = res


def _body(u_hbm, v_hbm, e_hbm, w_hbm, out_hbm,
          w_v, e_v, o_v, u_v0, v_v0, u_v1, v_v1,
          sem_u0, sem_v0, sem_u1, sem_v1):
    wid = lax.axis_index("s") * 2 + lax.axis_index("c")
    base = wid * PER_W

    pltpu.sync_copy(w_hbm, w_v)
    pltpu.sync_copy(e_hbm.at[pl.ds(base, PER_W)], e_v.at[pl.ds(0, PER_W)])
    e_v[pl.ds(PER_W, LANES)] = jnp.zeros((LANES,), jnp.int32)

    bufs = ((u_v0, v_v0, sem_u0, sem_v0), (u_v1, v_v1, sem_u1, sem_v1))

    def issue(c, buf):
        u_b, v_b, s_u, s_v = buf
        row0 = base + c * RCHUNK
        pltpu.async_copy(u_hbm.at[pl.ds(row0 * D, RCHUNK * D)], u_b, s_u)
        pltpu.async_copy(v_hbm.at[pl.ds(row0 * D, RCHUNK * D)], v_b, s_v)

    def wait(buf):
        u_b, v_b, s_u, s_v = buf
        pltpu.make_async_copy(u_hbm.at[pl.ds(0, RCHUNK * D)], u_b, s_u).wait()
        pltpu.make_async_copy(v_hbm.at[pl.ds(0, RCHUNK * D)], v_b, s_v).wait()

    def compute(c, buf):
        u_b, v_b, _, _ = buf
        for g in range(GROUPS):
            _group(u_b, v_b, w_v, e_v, o_v,
                   c * RCHUNK + g * LANES,
                   g * LANES,
                   c * RCHUNK + g * LANES)

    issue(0, bufs[0])
    issue(1, bufs[1])

    def pair_body(cc, _):
        c0 = cc * 2
        for p in range(2):
            wait(bufs[p])
            compute(c0 + p, bufs[p])

            @pl.when(c0 + p + 2 < NCHUNK)
            def _():
                issue(c0 + p + 2, bufs[p])
        return ()

    lax.fori_loop(0, NCHUNK // 2, pair_body, ())

    tail = NCHUNK * RCHUNK
    pltpu.sync_copy(u_hbm.at[pl.ds((base + tail) * D, REM * D)],
                    u_v0.at[pl.ds(0, REM * D)])
    pltpu.sync_copy(v_hbm.at[pl.ds((base + tail) * D, REM * D)],
                    v_v0.at[pl.ds(0, REM * D)])
    _group(u_v0, v_v0, w_v, e_v, o_v, tail, 0, tail)

    pltpu.sync_copy(o_v.at[pl.ds(0, PER_W)], out_hbm.at[pl.ds(base, PER_W)])


@jax.jit
def _distmult_sc(u_flat, v_flat, etype_i32, w_flat):
    mesh = plsc.VectorSubcoreMesh(core_axis_name="c", subcore_axis_name="s")
    return pl.kernel(
        _body,
        out_type=jax.ShapeDtypeStruct((B,), jnp.float32),
        mesh=mesh,
        scratch_types=[
            pltpu.VMEM((NUM_RELS * D,), jnp.float32),
            pltpu.VMEM((PER_W + 2 * LANES,), jnp.int32),
            pltpu.VMEM((PER_W + LANES,), jnp.float32),
            pltpu.VMEM((RCHUNK * D,), jnp.float32),
            pltpu.VMEM((RCHUNK * D,), jnp.float32),
            pltpu.VMEM((RCHUNK * D,), jnp.float32),
            pltpu.VMEM((RCHUNK * D,), jnp.float32),
            pltpu.SemaphoreType.DMA,
            pltpu.SemaphoreType.DMA,
            pltpu.SemaphoreType.DMA,
            pltpu.SemaphoreType.DMA,
        ],
    )(u_flat, v_flat, etype_i32, w_flat)


def kernel(u, v, etype_ids, w_relation):
    return _distmult_sc(
        u.reshape(-1),
        v.reshape(-1),
        etype_ids.astype(jnp.int32),
        w_relation.reshape(-1),
    )


# speedup vs baseline: 2.9341x; 2.2030x over previous
import jax
import jax.numpy as jnp
from jax import lax
from jax.experimental import pallas as pl
from jax.experimental.pallas import tpu as pltpu
from jax.experimental.pallas import tpu_sc as plsc

B = 160000
D = 256
NUM_RELS = 16
LANES = 16
NW = 32
PER_W = B // NW
RCHUNK = 48
GROUPS = RCHUNK // LANES
NCHUNK = 4992 // RCHUNK
REM = PER_W - NCHUNK * RCHUNK


def _row_score(u_v, v_v, w_v, e_v, row, brow):
    e = e_v[pl.ds(row, LANES)][0]
    acc = (u_v[brow, pl.ds(0, LANES)]
           * w_v[e, pl.ds(0, LANES)]
           * v_v[brow, pl.ds(0, LANES)])
    for k in range(1, D // LANES):
        acc = acc + (u_v[brow, pl.ds(k * LANES, LANES)]
                     * w_v[e, pl.ds(k * LANES, LANES)]
                     * v_v[brow, pl.ds(k * LANES, LANES)])
    lanes = lax.iota(jnp.int32, LANES)
    for step in (8, 4, 2, 1):
        acc = acc + acc.at[lanes ^ step].get(mode="promise_in_bounds")
    return acc


def _group(u_v, v_v, w_v, e_v, o_v, erow0, brow0, orow0):
    lanes = lax.iota(jnp.int32, LANES)

    def row_body(jj, res):
        s = _row_score(u_v, v_v, w_v, e_v, erow0 + jj, brow0 + jj)
        return jnp.where(lanes == jj, s, res)

    res = lax.fori_loop(0, LANES, row_body,
                        jnp.zeros((LANES,), jnp.float32), unroll=2)
    o_v[pl.ds(orow0, LANES)] = res


def _body(u_hbm, v_hbm, e_hbm, w_hbm, out_hbm,
          w_v, e_v, o_v, u_v0, v_v0, u_v1, v_v1,
          sem_u0, sem_v0, sem_u1, sem_v1):
    wid = lax.axis_index("s") * 2 + lax.axis_index("c")
    base = wid * PER_W

    pltpu.sync_copy(w_hbm, w_v)
    pltpu.sync_copy(e_hbm.at[pl.ds(base, PER_W)], e_v.at[pl.ds(0, PER_W)])
    e_v[pl.ds(PER_W, LANES)] = jnp.zeros((LANES,), jnp.int32)

    bufs = ((u_v0, v_v0, sem_u0, sem_v0), (u_v1, v_v1, sem_u1, sem_v1))

    def issue(c, buf):
        u_b, v_b, s_u, s_v = buf
        row0 = base + c * RCHUNK
        pltpu.async_copy(u_hbm.at[pl.ds(row0, RCHUNK)], u_b, s_u)
        pltpu.async_copy(v_hbm.at[pl.ds(row0, RCHUNK)], v_b, s_v)

    def wait(buf):
        u_b, v_b, s_u, s_v = buf
        pltpu.make_async_copy(u_hbm.at[pl.ds(0, RCHUNK)], u_b, s_u).wait()
        pltpu.make_async_copy(v_hbm.at[pl.ds(0, RCHUNK)], v_b, s_v).wait()

    def compute(c, buf):
        u_b, v_b, _, _ = buf
        for g in range(GROUPS):
            _group(u_b, v_b, w_v, e_v, o_v,
                   c * RCHUNK + g * LANES,
                   g * LANES,
                   c * RCHUNK + g * LANES)

    issue(0, bufs[0])
    issue(1, bufs[1])

    def pair_body(cc, _):
        c0 = cc * 2
        for p in range(2):
            wait(bufs[p])
            compute(c0 + p, bufs[p])

            @pl.when(c0 + p + 2 < NCHUNK)
            def _():
                issue(c0 + p + 2, bufs[p])
        return ()

    lax.fori_loop(0, NCHUNK // 2, pair_body, ())

    tail = NCHUNK * RCHUNK
    pltpu.sync_copy(u_hbm.at[pl.ds(base + tail, REM)],
                    u_v0.at[pl.ds(0, REM)])
    pltpu.sync_copy(v_hbm.at[pl.ds(base + tail, REM)],
                    v_v0.at[pl.ds(0, REM)])
    _group(u_v0, v_v0, w_v, e_v, o_v, tail, 0, tail)

    pltpu.sync_copy(o_v.at[pl.ds(0, PER_W)], out_hbm.at[pl.ds(base, PER_W)])


@jax.jit
def _distmult_sc(u, v, etype_i32, w):
    mesh = plsc.VectorSubcoreMesh(core_axis_name="c", subcore_axis_name="s")
    return pl.kernel(
        _body,
        out_type=jax.ShapeDtypeStruct((B,), jnp.float32),
        mesh=mesh,
        compiler_params=pltpu.CompilerParams(use_tc_tiling_on_sc=True),
        scratch_types=[
            pltpu.VMEM((NUM_RELS, D), jnp.float32),
            pltpu.VMEM((PER_W + 2 * LANES,), jnp.int32),
            pltpu.VMEM((PER_W + LANES,), jnp.float32),
            pltpu.VMEM((RCHUNK, D), jnp.float32),
            pltpu.VMEM((RCHUNK, D), jnp.float32),
            pltpu.VMEM((RCHUNK, D), jnp.float32),
            pltpu.VMEM((RCHUNK, D), jnp.float32),
            pltpu.SemaphoreType.DMA,
            pltpu.SemaphoreType.DMA,
            pltpu.SemaphoreType.DMA,
            pltpu.SemaphoreType.DMA,
        ],
    )(u, v, etype_i32, w)


def kernel(u, v, etype_ids, w_relation):
    return _distmult_sc(u, v, etype_ids.astype(jnp.int32), w_relation)


# speedup vs baseline: 3.4819x; 1.1867x over previous
import jax
import jax.numpy as jnp
from jax import lax
from jax.experimental import pallas as pl
from jax.experimental.pallas import tpu as pltpu
from jax.experimental.pallas import tpu_sc as plsc

B = 160000
D = 256
NUM_RELS = 16
LANES = 16

TC_BLOCK = 1024
B_TC = 78 * TC_BLOCK
B_SC = B - B_TC

NW = 32
PER_W = B_SC // NW
RCHUNK = 48
GROUPS = RCHUNK // LANES
NCHUNK = PER_W // RCHUNK
REM = PER_W - NCHUNK * RCHUNK


def _row_score(u_v, v_v, w_v, e_v, row, brow):
    e = e_v[pl.ds(row, LANES)][0]
    acc = (u_v[brow, pl.ds(0, LANES)]
           * w_v[e, pl.ds(0, LANES)]
           * v_v[brow, pl.ds(0, LANES)])
    for k in range(1, D // LANES):
        acc = acc + (u_v[brow, pl.ds(k * LANES, LANES)]
                     * w_v[e, pl.ds(k * LANES, LANES)]
                     * v_v[brow, pl.ds(k * LANES, LANES)])
    lanes = lax.iota(jnp.int32, LANES)
    for step in (8, 4, 2, 1):
        acc = acc + acc.at[lanes ^ step].get(mode="promise_in_bounds")
    return acc


def _group(u_v, v_v, w_v, e_v, o_v, erow0, brow0, orow0):
    lanes = lax.iota(jnp.int32, LANES)

    def row_body(jj, res):
        s = _row_score(u_v, v_v, w_v, e_v, erow0 + jj, brow0 + jj)
        return jnp.where(lanes == jj, s, res)

    res = lax.fori_loop(0, LANES, row_body,
                        jnp.zeros((LANES,), jnp.float32), unroll=2)
    o_v[pl.ds(orow0, LANES)] = res


def _sc_body(u_hbm, v_hbm, e_hbm, w_hbm, out_hbm,
             w_v, e_v, o_v, u_v0, v_v0, u_v1, v_v1,
             sem_u0, sem_v0, sem_u1, sem_v1):
    wid = lax.axis_index("s") * 2 + lax.axis_index("c")
    base = B_TC + wid * PER_W

    pltpu.sync_copy(w_hbm, w_v)
    pltpu.sync_copy(e_hbm.at[pl.ds(base, PER_W)], e_v.at[pl.ds(0, PER_W)])
    e_v[pl.ds(PER_W, LANES)] = jnp.zeros((LANES,), jnp.int32)

    bufs = ((u_v0, v_v0, sem_u0, sem_v0), (u_v1, v_v1, sem_u1, sem_v1))

    def issue(c, buf):
        u_b, v_b, s_u, s_v = buf
        row0 = base + c * RCHUNK
        pltpu.async_copy(u_hbm.at[pl.ds(row0, RCHUNK)], u_b, s_u)
        pltpu.async_copy(v_hbm.at[pl.ds(row0, RCHUNK)], v_b, s_v)

    def wait(buf):
        u_b, v_b, s_u, s_v = buf
        pltpu.make_async_copy(u_hbm.at[pl.ds(0, RCHUNK)], u_b, s_u).wait()
        pltpu.make_async_copy(v_hbm.at[pl.ds(0, RCHUNK)], v_b, s_v).wait()

    def compute(c, buf):
        u_b, v_b, _, _ = buf
        for g in range(GROUPS):
            _group(u_b, v_b, w_v, e_v, o_v,
                   c * RCHUNK + g * LANES,
                   g * LANES,
                   c * RCHUNK + g * LANES)

    issue(0, bufs[0])
    issue(1, bufs[1])

    def pair_body(cc, _):
        c0 = cc * 2
        for p in range(2):
            wait(bufs[p])
            compute(c0 + p, bufs[p])

            @pl.when(c0 + p + 2 < NCHUNK)
            def _():
                issue(c0 + p + 2, bufs[p])
        return ()

    lax.fori_loop(0, NCHUNK // 2, pair_body, ())

    tail = NCHUNK * RCHUNK
    pltpu.sync_copy(u_hbm.at[pl.ds(base + tail, REM)],
                    u_v0.at[pl.ds(0, REM)])
    pltpu.sync_copy(v_hbm.at[pl.ds(base + tail, REM)],
                    v_v0.at[pl.ds(0, REM)])
    _group(u_v0, v_v0, w_v, e_v, o_v, tail, 0, tail)

    pltpu.sync_copy(o_v.at[pl.ds(0, PER_W)],
                    out_hbm.at[pl.ds(wid * PER_W, PER_W)])


def _sc_call(u, v, etype_i32, w):
    mesh = plsc.VectorSubcoreMesh(core_axis_name="c", subcore_axis_name="s")
    return pl.kernel(
        _sc_body,
        out_type=jax.ShapeDtypeStruct((B_SC,), jnp.float32),
        mesh=mesh,
        compiler_params=pltpu.CompilerParams(use_tc_tiling_on_sc=True),
        scratch_types=[
            pltpu.VMEM((NUM_RELS, D), jnp.float32),
            pltpu.VMEM((PER_W + 2 * LANES,), jnp.int32),
            pltpu.VMEM((PER_W + LANES,), jnp.float32),
            pltpu.VMEM((RCHUNK, D), jnp.float32),
            pltpu.VMEM((RCHUNK, D), jnp.float32),
            pltpu.VMEM((RCHUNK, D), jnp.float32),
            pltpu.VMEM((RCHUNK, D), jnp.float32),
            pltpu.SemaphoreType.DMA,
            pltpu.SemaphoreType.DMA,
            pltpu.SemaphoreType.DMA,
            pltpu.SemaphoreType.DMA,
        ],
    )(u, v, etype_i32, w)


def _tc_body(u_ref, v_ref, oh_ref, w_ref, o_ref):
    uv = u_ref[...] * v_ref[...]
    p = lax.dot_general(uv, w_ref[...],
                        (((1,), (1,)), ((), ())),
                        preferred_element_type=jnp.float32)
    o_ref[...] = jnp.sum(oh_ref[...] * p, axis=1)


def _tc_call(u, v, onehot, w):
    nblk = B_TC // TC_BLOCK
    return pl.pallas_call(
        _tc_body,
        grid=(nblk,),
        in_specs=[
            pl.BlockSpec((TC_BLOCK, D), lambda j: (j, 0)),
            pl.BlockSpec((TC_BLOCK, D), lambda j: (j, 0)),
            pl.BlockSpec((TC_BLOCK, NUM_RELS), lambda j: (j, 0)),
            pl.BlockSpec((NUM_RELS, D), lambda j: (0, 0)),
        ],
        out_specs=pl.BlockSpec((TC_BLOCK,), lambda j: (j,)),
        out_shape=jax.ShapeDtypeStruct((B_TC,), jnp.float32),
    )(u, v, onehot, w)


@jax.jit
def _distmult(u, v, etype_i32, w):
    sc_out = _sc_call(u, v, etype_i32, w)
    onehot = (etype_i32[:B_TC, None]
              == jnp.arange(NUM_RELS, dtype=jnp.int32)[None, :]
              ).astype(jnp.float32)
    tc_out = _tc_call(u, v, onehot, w)
    return jnp.concatenate([tc_out, sc_out], axis=0)


def kernel(u, v, etype_ids, w_relation):
    return _distmult(u, v, etype_ids.astype(jnp.int32), w_relation)
